# Initial kernel scaffold; baseline (speedup 1.0000x reference)
import jax
import jax.numpy as jnp
from jax.experimental import pallas as pl


def kernel(x, embed_mat):
    raise NotImplementedError("write your pallas kernel here")


# speedup vs baseline: 1.8389x; 1.8389x over previous
import functools
import math

import jax
import jax.numpy as jnp
from jax import lax
from jax.experimental import pallas as pl
from jax.experimental.pallas import tpu as pltpu
from jax.experimental.pallas import tpu_sc as plsc

L = 16


def _rsqrt_nr(s):
    i = plsc.bitcast(s, jnp.int32)
    y = plsc.bitcast(jnp.int32(0x5F3759DF) - (i >> 1), jnp.float32)
    for _ in range(3):
        y = y * (1.5 - 0.5 * s * y * y)
    return y


def _emb_body(table_hbm, idx_hbm, out_hbm, idx_v, buf, sem,
              *, b_per_w, ch, n_ch, d, nc, scale):
    wid = lax.axis_index("s") * nc + lax.axis_index("c")
    base = wid * b_per_w
    pltpu.sync_copy(idx_hbm.at[pl.ds(base, b_per_w)], idx_v)

    def chunk(c, carry):
        off = c * ch
        pltpu.async_copy(table_hbm.at[idx_v.at[pl.ds(off, ch)]], buf, sem).wait()

        def row(r, carry2):
            acc = jnp.zeros((L,), jnp.float32)
            vs = []
            for j in range(d // L):
                v = buf[r, pl.ds(j * L, L)]
                vs.append(v)
                acc = acc + v * v
            s = jnp.full((L,), jnp.sum(acc), jnp.float32)
            y = _rsqrt_nr(s) * scale
            for j in range(d // L):
                buf[r, pl.ds(j * L, L)] = vs[j] * y
            return carry2

        lax.fori_loop(0, ch, row, 0)
        pltpu.sync_copy(buf, out_hbm.at[pl.ds(base + off, ch)])
        return carry

    lax.fori_loop(0, n_ch, chunk, 0)


def kernel(x, embed_mat):
    b, s_len = x.shape
    v, d = embed_mat.shape
    n = b * s_len
    info = plsc.get_sparse_core_info()
    nc, ns = info.num_cores, info.num_subcores
    nw = nc * ns
    b_per_w = n // nw
    ch = 128
    n_ch = b_per_w // ch
    scale = math.sqrt(d)

    mesh = plsc.VectorSubcoreMesh(core_axis_name="c", subcore_axis_name="s")
    emb = functools.partial(
        pl.kernel,
        mesh=mesh,
        compiler_params=pltpu.CompilerParams(needs_layout_passes=False),
        out_type=jax.ShapeDtypeStruct((n, d), jnp.float32),
        scratch_types=[
            pltpu.VMEM((b_per_w,), jnp.int32),
            pltpu.VMEM((ch, d), jnp.float32),
            pltpu.SemaphoreType.DMA,
        ],
    )(functools.partial(_emb_body, b_per_w=b_per_w, ch=ch, n_ch=n_ch,
                        d=d, nc=nc, scale=scale))

    out = emb(embed_mat, x.reshape(n))
    return out.reshape(b, s_len, d)


# speedup vs baseline: 3.2419x; 1.7629x over previous
import functools
import math

import jax
import jax.numpy as jnp
from jax import lax
from jax.experimental import pallas as pl
from jax.experimental.pallas import tpu as pltpu
from jax.experimental.pallas import tpu_sc as plsc

L = 16


def _rsqrt_nr(s):
    i = plsc.bitcast(s, jnp.int32)
    y = plsc.bitcast(jnp.int32(0x5F3759DF) - (i >> 1), jnp.float32)
    for _ in range(3):
        y = y * (1.5 - 0.5 * s * y * y)
    return y


def _normalize(gbuf, obuf, ch, d, scale):
    iota = lax.iota(jnp.int32, L)

    def group(g, carry):
        r0 = g * L
        tot = jnp.zeros((L,), jnp.float32)
        for rp in range(L):
            sq = [None] * (d // L)
            for j in range(d // L):
                v = gbuf[r0 + rp, pl.ds(j * L, L)]
                sq[j] = v * v
            while len(sq) > 1:
                sq = [sq[i] + sq[i + 1] for i in range(0, len(sq) - 1, 2)] + (
                    [sq[-1]] if len(sq) % 2 else [])
            s = jnp.sum(sq[0])
            tot = jnp.where(iota == rp, s, tot)
        y = _rsqrt_nr(tot) * scale
        for rp in range(L):
            yv = jnp.full((L,), y[rp], jnp.float32)
            for j in range(d // L):
                obuf[r0 + rp, pl.ds(j * L, L)] = (
                    gbuf[r0 + rp, pl.ds(j * L, L)] * yv)
        return carry

    lax.fori_loop(0, ch // L, group, 0)


def _emb_body(table_hbm, idx_hbm, out_hbm, idx_v, gbuf0, gbuf1, obuf0, obuf1,
              gsem0, gsem1, osem0, osem1,
              *, b_per_w, ch, n_ch, d, nc, scale):
    wid = lax.axis_index("s") * nc + lax.axis_index("c")
    base = wid * b_per_w
    pltpu.sync_copy(idx_hbm.at[pl.ds(base, b_per_w)], idx_v)

    def gather(c, buf, sem):
        return pltpu.async_copy(table_hbm.at[idx_v.at[pl.ds(c * ch, ch)]],
                                buf, sem)

    def writeback(c, buf, sem):
        return pltpu.async_copy(buf, out_hbm.at[pl.ds(base + c * ch, ch)], sem)

    gather(0, gbuf0, gsem0)

    def pair(c2, carry):
        c0 = 2 * c2
        gather(c0 + 1, gbuf1, gsem1)
        pltpu.make_async_copy(table_hbm.at[idx_v.at[pl.ds(c0 * ch, ch)]],
                              gbuf0, gsem0).wait()

        @pl.when(c2 > 0)
        def _():
            pltpu.make_async_copy(
                obuf0, out_hbm.at[pl.ds(base + (c0 - 2) * ch, ch)],
                osem0).wait()

        _normalize(gbuf0, obuf0, ch, d, scale)
        writeback(c0, obuf0, osem0)

        @pl.when(c2 < n_ch // 2 - 1)
        def _():
            gather(c0 + 2, gbuf0, gsem0)

        pltpu.make_async_copy(table_hbm.at[idx_v.at[pl.ds((c0 + 1) * ch, ch)]],
                              gbuf1, gsem1).wait()

        @pl.when(c2 > 0)
        def _():
            pltpu.make_async_copy(
                obuf1, out_hbm.at[pl.ds(base + (c0 - 1) * ch, ch)],
                osem1).wait()

        _normalize(gbuf1, obuf1, ch, d, scale)
        writeback(c0 + 1, obuf1, osem1)
        return carry

    lax.fori_loop(0, n_ch // 2, pair, 0)
    pltpu.make_async_copy(obuf0, out_hbm.at[pl.ds(base + (n_ch - 2) * ch, ch)],
                          osem0).wait()
    pltpu.make_async_copy(obuf1, out_hbm.at[pl.ds(base + (n_ch - 1) * ch, ch)],
                          osem1).wait()


def kernel(x, embed_mat):
    b, s_len = x.shape
    v, d = embed_mat.shape
    n = b * s_len
    info = plsc.get_sparse_core_info()
    nc, ns = info.num_cores, info.num_subcores
    nw = nc * ns
    b_per_w = n // nw
    ch = 128
    n_ch = b_per_w // ch
    scale = math.sqrt(d)

    mesh = plsc.VectorSubcoreMesh(core_axis_name="c", subcore_axis_name="s")
    emb = functools.partial(
        pl.kernel,
        mesh=mesh,
        compiler_params=pltpu.CompilerParams(needs_layout_passes=False),
        out_type=jax.ShapeDtypeStruct((n, d), jnp.float32),
        scratch_types=[
            pltpu.VMEM((b_per_w,), jnp.int32),
            pltpu.VMEM((ch, d), jnp.float32),
            pltpu.VMEM((ch, d), jnp.float32),
            pltpu.VMEM((ch, d), jnp.float32),
            pltpu.VMEM((ch, d), jnp.float32),
            pltpu.SemaphoreType.DMA,
            pltpu.SemaphoreType.DMA,
            pltpu.SemaphoreType.DMA,
            pltpu.SemaphoreType.DMA,
        ],
    )(functools.partial(_emb_body, b_per_w=b_per_w, ch=ch, n_ch=n_ch,
                        d=d, nc=nc, scale=scale))

    out = emb(embed_mat, x.reshape(n))
    return out.reshape(b, s_len, d)
